# baseline (speedup 1.0000x reference)
import functools

import jax
import jax.numpy as jnp
from jax import lax
from jax.experimental import pallas as pl
from jax.experimental.pallas import tpu as pltpu
from jax.experimental.pallas import tpu_sc as plsc

N = 10000
E = 320000
D = 128

NC = 2
NS = 16
NW = NC * NS
EPW = E // NW

NPD = 10240
RPD = NPD // 16

CHB = 128
NCHB = 80
EPWP = NCHB * CHB
G = 8
NG = NCHB // G

NP = 10112
RPT = NP // NS

_mesh = plsc.VectorSubcoreMesh(core_axis_name="c", subcore_axis_name="s")


@functools.partial(
    pl.kernel,
    out_type=jax.ShapeDtypeStruct((NC, 2, NPD), jnp.float32),
    mesh=_mesh,
    scratch_types=[
        pltpu.VMEM((NCHB, CHB), jnp.int32),
        pltpu.VMEM((NCHB, CHB), jnp.int32),
        pltpu.VMEM((CHB,), jnp.float32),
        pltpu.VMEM_SHARED((NPD,), jnp.float32),
        pltpu.VMEM_SHARED((NPD,), jnp.float32),
    ],
)
def _sc_degrees(srcr_hbm, dstr_hbm, ones_hbm, zeros_hbm, degp_hbm,
                sidx, didx, ones_v, dsrc, ddst):
    c = lax.axis_index("c")
    s = lax.axis_index("s")
    wid = s * NC + c
    base = s * RPD
    pltpu.sync_copy(zeros_hbm, dsrc.at[pl.ds(base, RPD)])
    pltpu.sync_copy(zeros_hbm, ddst.at[pl.ds(base, RPD)])
    pltpu.sync_copy(ones_hbm, ones_v)
    pltpu.sync_copy(srcr_hbm.at[wid], sidx)
    pltpu.sync_copy(dstr_hbm.at[wid], didx)
    plsc.subcore_barrier()

    @pl.loop(0, NCHB)
    def _chunk(j):
        pltpu.sync_copy(ones_v, dsrc.at[sidx.at[j]], add=True)
        pltpu.sync_copy(ones_v, ddst.at[didx.at[j]], add=True)

    plsc.subcore_barrier()
    pltpu.sync_copy(dsrc.at[pl.ds(base, RPD)], degp_hbm.at[c, 0, pl.ds(base, RPD)])
    pltpu.sync_copy(ddst.at[pl.ds(base, RPD)], degp_hbm.at[c, 1, pl.ds(base, RPD)])


@functools.partial(
    pl.kernel,
    out_type=jax.ShapeDtypeStruct((NC, NP, D), jnp.float32),
    mesh=_mesh,
    scratch_types=[
        pltpu.VMEM((2, G, CHB), jnp.int32),
        pltpu.VMEM((NCHB, CHB), jnp.int32),
        pltpu.VMEM((2, CHB, D), jnp.float32),
        pltpu.VMEM_SHARED((NP, D), jnp.float32),
        pltpu.SemaphoreType.DMA((2,)),
        pltpu.SemaphoreType.DMA((2,)),
        pltpu.SemaphoreType.DMA((2,)),
    ],
)
def _sc_spmm(featn_hbm, srcg_hbm, dstr_hbm, zrows_hbm, part_hbm,
             sbuf, didx, rows, accum, gsem, isem, ssem):
    c = lax.axis_index("c")
    s = lax.axis_index("s")
    wid = s * NC + c
    base = s * RPT

    def start_idx(g, gb):
        pltpu.async_copy(srcg_hbm.at[wid, g], sbuf.at[gb], isem.at[gb])

    def wait_idx(g, gb):
        pltpu.make_async_copy(srcg_hbm.at[wid, g], sbuf.at[gb],
                              isem.at[gb]).wait()

    def start_gather(gb, k, rb):
        pltpu.async_copy(featn_hbm.at[sbuf.at[gb, k]], rows.at[rb],
                         gsem.at[rb])

    def wait_gather(gb, k, rb):
        pltpu.make_async_copy(featn_hbm.at[sbuf.at[gb, k]], rows.at[rb],
                              gsem.at[rb]).wait()

    def start_scatter(t, rb):
        pltpu.async_copy(rows.at[rb], accum.at[didx.at[t]], ssem.at[rb],
                         add=True)

    def wait_scatter(t, rb):
        pltpu.make_async_copy(rows.at[rb], accum.at[didx.at[t]],
                              ssem.at[rb]).wait()

    pltpu.sync_copy(zrows_hbm, accum.at[pl.ds(base, RPT)])
    pltpu.sync_copy(dstr_hbm.at[wid], didx)
    start_idx(0, 0)
    plsc.subcore_barrier()

    wait_idx(0, 0)
    start_idx(1, 1)
    start_gather(0, 0, 0)
    start_gather(0, 1, 1)

    @pl.loop(0, NCHB)
    def _chunk(t):
        rb = lax.rem(t, 2)
        g = lax.div(t, G)
        k = lax.rem(t, G)
        gb = lax.rem(g, 2)

        wait_gather(gb, k, rb)
        start_scatter(t, rb)

        tp = t - 1

        @pl.when(tp >= 0)
        def _():
            rbp = 1 - rb
            wait_scatter(tp, rbp)
            tg = t + 1
            gn = lax.div(tg, G)
            kn = lax.rem(tg, G)
            gbn = lax.rem(gn, 2)

            @pl.when(tg < NCHB)
            def _():
                @pl.when(kn == 0)
                def _():
                    wait_idx(gn, gbn)

                    @pl.when(gn + 1 < NG)
                    def _():
                        start_idx(gn + 1, 1 - gbn)

                start_gather(gbn, kn, rbp)

    wait_scatter(NCHB - 1, (NCHB - 1) % 2)
    plsc.subcore_barrier()
    pltpu.sync_copy(accum.at[pl.ds(base, RPT)], part_hbm.at[c, pl.ds(base, RPT)])


def _tc_scale(feat, deg):
    def body(f_ref, d_ref, o_ref):
        o_ref[...] = f_ref[...] * lax.rsqrt(jnp.maximum(d_ref[...], 1.0))

    blk = 1000
    return pl.pallas_call(
        body,
        grid=(N // blk,),
        in_specs=[
            pl.BlockSpec((blk, D), lambda i: (i, 0)),
            pl.BlockSpec((blk, 1), lambda i: (i, 0)),
        ],
        out_specs=pl.BlockSpec((blk, D), lambda i: (i, 0)),
        out_shape=jax.ShapeDtypeStruct((N, D), jnp.float32),
    )(feat, deg)


def _tc_matmul_scale(parts, weight, deg):
    def body(p_ref, w_ref, d_ref, o_ref):
        ssum = p_ref[0] + p_ref[1]
        r = jnp.dot(ssum, w_ref[...], preferred_element_type=jnp.float32)
        o_ref[...] = r * lax.rsqrt(jnp.maximum(d_ref[...], 1.0))

    blk = 1000
    return pl.pallas_call(
        body,
        grid=(N // blk,),
        in_specs=[
            pl.BlockSpec((NC, blk, D), lambda i: (0, i, 0)),
            pl.BlockSpec((D, D), lambda i: (0, 0)),
            pl.BlockSpec((blk, 1), lambda i: (i, 0)),
        ],
        out_specs=pl.BlockSpec((blk, D), lambda i: (i, 0)),
        out_shape=jax.ShapeDtypeStruct((N, D), jnp.float32),
    )(parts, weight, deg)


def kernel(feat, edge_index, weight):
    src = edge_index[0].astype(jnp.int32)
    dst = edge_index[1].astype(jnp.int32)
    pad = EPWP - EPW
    srcd = jnp.pad(src.reshape(NW, EPW), ((0, 0), (0, pad)),
                   constant_values=NPD - 1).reshape(NW, NCHB, CHB)
    dstp = jnp.pad(dst.reshape(NW, EPW), ((0, 0), (0, pad)),
                   constant_values=NP - 1).reshape(NW, NCHB, CHB)

    ones1 = jnp.ones((CHB,), jnp.float32)
    zeros1 = jnp.zeros((RPD,), jnp.float32)
    degp = _sc_degrees(srcd, dstp, ones1, zeros1)
    deg_src = (degp[0, 0, :N] + degp[1, 0, :N]).reshape(N, 1)
    deg_dst = (degp[0, 1, :N] + degp[1, 1, :N]).reshape(N, 1)

    featn = _tc_scale(feat, deg_src)

    srcg = jnp.pad(src.reshape(NW, EPW), ((0, 0), (0, pad)),
                   constant_values=0).reshape(NW, NG, G, CHB)
    zrows = jnp.zeros((RPT, D), jnp.float32)
    parts = _sc_spmm(featn, srcg, dstp, zrows)[:, :N]

    return _tc_matmul_scale(parts, weight, deg_dst)


# speedup vs baseline: 1.0600x; 1.0600x over previous
import functools

import jax
import jax.numpy as jnp
from jax import lax
from jax.experimental import pallas as pl
from jax.experimental.pallas import tpu as pltpu
from jax.experimental.pallas import tpu_sc as plsc

N = 10000
E = 320000
D = 128

NC = 2
NS = 16
NW = NC * NS
EPW = E // NW

NPD = 10240
RPD = NPD // 16

CHB = 128
NCHB = 80
EPWP = NCHB * CHB
G = 8
NG = NCHB // G

NP = 10112
RPT = NP // NS

_mesh = plsc.VectorSubcoreMesh(core_axis_name="c", subcore_axis_name="s")


@functools.partial(
    pl.kernel,
    out_type=jax.ShapeDtypeStruct((NC, 2, NPD), jnp.float32),
    mesh=_mesh,
    scratch_types=[
        pltpu.VMEM((NCHB, CHB), jnp.int32),
        pltpu.VMEM((NCHB, CHB), jnp.int32),
        pltpu.VMEM((CHB,), jnp.float32),
        pltpu.VMEM_SHARED((NPD,), jnp.float32),
        pltpu.VMEM_SHARED((NPD,), jnp.float32),
    ],
)
def _sc_degrees(srcr_hbm, dstr_hbm, ones_hbm, zeros_hbm, degp_hbm,
                sidx, didx, ones_v, dsrc, ddst):
    c = lax.axis_index("c")
    s = lax.axis_index("s")
    wid = s * NC + c
    base = s * RPD
    pltpu.sync_copy(zeros_hbm, dsrc.at[pl.ds(base, RPD)])
    pltpu.sync_copy(zeros_hbm, ddst.at[pl.ds(base, RPD)])
    pltpu.sync_copy(ones_hbm, ones_v)
    pltpu.sync_copy(srcr_hbm.at[wid], sidx)
    pltpu.sync_copy(dstr_hbm.at[wid], didx)
    plsc.subcore_barrier()

    @pl.loop(0, NCHB)
    def _chunk(j):
        pltpu.sync_copy(ones_v, dsrc.at[sidx.at[j]], add=True)
        pltpu.sync_copy(ones_v, ddst.at[didx.at[j]], add=True)

    plsc.subcore_barrier()
    pltpu.sync_copy(dsrc.at[pl.ds(base, RPD)], degp_hbm.at[c, 0, pl.ds(base, RPD)])
    pltpu.sync_copy(ddst.at[pl.ds(base, RPD)], degp_hbm.at[c, 1, pl.ds(base, RPD)])


@functools.partial(
    pl.kernel,
    out_type=jax.ShapeDtypeStruct((NC, NP, D), jnp.float32),
    mesh=_mesh,
    scratch_types=[
        pltpu.VMEM((2, G, CHB), jnp.int32),
        pltpu.VMEM((NCHB, CHB), jnp.int32),
        pltpu.VMEM((2, CHB, D), jnp.float32),
        pltpu.VMEM_SHARED((NP, D), jnp.float32),
        pltpu.SemaphoreType.DMA((2,)),
        pltpu.SemaphoreType.DMA((2,)),
    ],
)
def _sc_spmm(featn_hbm, srcg_hbm, dstr_hbm, zrows_hbm, part_hbm,
             sbuf, didx, rows, accum, gsem, isem):
    c = lax.axis_index("c")
    s = lax.axis_index("s")
    wid = s * NC + c
    base = s * RPT

    def start_idx(g, gb):
        pltpu.async_copy(srcg_hbm.at[wid, g], sbuf.at[gb], isem.at[gb])

    def wait_idx(g, gb):
        pltpu.make_async_copy(srcg_hbm.at[wid, g], sbuf.at[gb],
                              isem.at[gb]).wait()

    def start_gather(gb, k, rb):
        pltpu.async_copy(featn_hbm.at[sbuf.at[gb, k]], rows.at[rb],
                         gsem.at[rb])

    def wait_gather(gb, k, rb):
        pltpu.make_async_copy(featn_hbm.at[sbuf.at[gb, k]], rows.at[rb],
                              gsem.at[rb]).wait()

    pltpu.sync_copy(zrows_hbm, accum.at[pl.ds(base, RPT)])
    pltpu.sync_copy(dstr_hbm.at[wid], didx)
    start_idx(0, 0)
    plsc.subcore_barrier()

    wait_idx(0, 0)
    start_idx(1, 1)
    start_gather(0, 0, 0)

    @pl.loop(0, NCHB)
    def _chunk(t):
        rb = lax.rem(t, 2)
        g = lax.div(t, G)
        k = lax.rem(t, G)
        gb = lax.rem(g, 2)
        tn = t + 1
        gn = lax.div(tn, G)
        kn = lax.rem(tn, G)
        rbn = lax.rem(tn, 2)
        gbn = lax.rem(gn, 2)

        @pl.when(jnp.logical_and(tn < NCHB, kn == 0))
        def _():
            wait_idx(gn, gbn)

            @pl.when(gn + 1 < NG)
            def _():
                start_idx(gn + 1, 1 - gbn)

        @pl.when(tn < NCHB)
        def _():
            start_gather(gbn, kn, rbn)

        wait_gather(gb, k, rb)
        pltpu.sync_copy(rows.at[rb], accum.at[didx.at[t]], add=True)

    plsc.subcore_barrier()
    pltpu.sync_copy(accum.at[pl.ds(base, RPT)], part_hbm.at[c, pl.ds(base, RPT)])


def _tc_scale(feat, deg):
    def body(f_ref, d_ref, o_ref):
        o_ref[...] = f_ref[...] * lax.rsqrt(jnp.maximum(d_ref[...], 1.0))

    blk = 1000
    return pl.pallas_call(
        body,
        grid=(N // blk,),
        in_specs=[
            pl.BlockSpec((blk, D), lambda i: (i, 0)),
            pl.BlockSpec((blk, 1), lambda i: (i, 0)),
        ],
        out_specs=pl.BlockSpec((blk, D), lambda i: (i, 0)),
        out_shape=jax.ShapeDtypeStruct((N, D), jnp.float32),
    )(feat, deg)


def _tc_matmul_scale(parts, weight, deg):
    def body(p_ref, w_ref, d_ref, o_ref):
        ssum = p_ref[0] + p_ref[1]
        r = jnp.dot(ssum, w_ref[...], preferred_element_type=jnp.float32)
        o_ref[...] = r * lax.rsqrt(jnp.maximum(d_ref[...], 1.0))

    blk = 1000
    return pl.pallas_call(
        body,
        grid=(N // blk,),
        in_specs=[
            pl.BlockSpec((NC, blk, D), lambda i: (0, i, 0)),
            pl.BlockSpec((D, D), lambda i: (0, 0)),
            pl.BlockSpec((blk, 1), lambda i: (i, 0)),
        ],
        out_specs=pl.BlockSpec((blk, D), lambda i: (i, 0)),
        out_shape=jax.ShapeDtypeStruct((N, D), jnp.float32),
    )(parts, weight, deg)


def kernel(feat, edge_index, weight):
    src = edge_index[0].astype(jnp.int32)
    dst = edge_index[1].astype(jnp.int32)
    pad = EPWP - EPW
    srcd = jnp.pad(src.reshape(NW, EPW), ((0, 0), (0, pad)),
                   constant_values=NPD - 1).reshape(NW, NCHB, CHB)
    dstp = jnp.pad(dst.reshape(NW, EPW), ((0, 0), (0, pad)),
                   constant_values=NP - 1).reshape(NW, NCHB, CHB)

    ones1 = jnp.ones((CHB,), jnp.float32)
    zeros1 = jnp.zeros((RPD,), jnp.float32)
    degp = _sc_degrees(srcd, dstp, ones1, zeros1)
    deg_src = (degp[0, 0, :N] + degp[1, 0, :N]).reshape(N, 1)
    deg_dst = (degp[0, 1, :N] + degp[1, 1, :N]).reshape(N, 1)

    featn = _tc_scale(feat, deg_src)

    srcg = jnp.pad(src.reshape(NW, EPW), ((0, 0), (0, pad)),
                   constant_values=0).reshape(NW, NG, G, CHB)
    zrows = jnp.zeros((RPT, D), jnp.float32)
    parts = _sc_spmm(featn, srcg, dstp, zrows)

    return _tc_matmul_scale(parts, weight, deg_dst)


# speedup vs baseline: 1.0602x; 1.0002x over previous
import functools

import jax
import jax.numpy as jnp
from jax import lax
from jax.experimental import pallas as pl
from jax.experimental.pallas import tpu as pltpu
from jax.experimental.pallas import tpu_sc as plsc

N = 10000
E = 320000
D = 128

NC = 2
NS = 16
NW = NC * NS
EPW = E // NW

NPD = 10240
RPD = NPD // 16

CHB = 128
NCHB = 80
EPWP = NCHB * CHB
G = 8
NG = NCHB // G

NP = 10112
RPT = NP // NS

_mesh = plsc.VectorSubcoreMesh(core_axis_name="c", subcore_axis_name="s")


@functools.partial(
    pl.kernel,
    out_type=jax.ShapeDtypeStruct((NC, 2, NPD), jnp.float32),
    mesh=_mesh,
    scratch_types=[
        pltpu.VMEM((NCHB, CHB), jnp.int32),
        pltpu.VMEM((NCHB, CHB), jnp.int32),
        pltpu.VMEM((CHB,), jnp.float32),
        pltpu.VMEM_SHARED((NPD,), jnp.float32),
        pltpu.VMEM_SHARED((NPD,), jnp.float32),
    ],
)
def _sc_degrees(srcr_hbm, dstr_hbm, ones_hbm, zeros_hbm, degp_hbm,
                sidx, didx, ones_v, dsrc, ddst):
    c = lax.axis_index("c")
    s = lax.axis_index("s")
    wid = s * NC + c
    base = s * RPD
    pltpu.sync_copy(zeros_hbm, dsrc.at[pl.ds(base, RPD)])
    pltpu.sync_copy(zeros_hbm, ddst.at[pl.ds(base, RPD)])
    pltpu.sync_copy(ones_hbm, ones_v)
    pltpu.sync_copy(srcr_hbm.at[wid], sidx)
    pltpu.sync_copy(dstr_hbm.at[wid], didx)
    plsc.subcore_barrier()

    @pl.loop(0, NCHB)
    def _chunk(j):
        pltpu.sync_copy(ones_v, dsrc.at[sidx.at[j]], add=True)
        pltpu.sync_copy(ones_v, ddst.at[didx.at[j]], add=True)

    plsc.subcore_barrier()
    pltpu.sync_copy(dsrc.at[pl.ds(base, RPD)], degp_hbm.at[c, 0, pl.ds(base, RPD)])
    pltpu.sync_copy(ddst.at[pl.ds(base, RPD)], degp_hbm.at[c, 1, pl.ds(base, RPD)])


@functools.partial(
    pl.kernel,
    out_type=jax.ShapeDtypeStruct((NC, NP, D), jnp.float32),
    mesh=_mesh,
    scratch_types=[
        pltpu.VMEM((2, G, CHB), jnp.int32),
        pltpu.VMEM((NCHB, CHB), jnp.int32),
        pltpu.VMEM((2, CHB, D), jnp.float32),
        pltpu.VMEM_SHARED((NP, D), jnp.float32),
        pltpu.SemaphoreType.DMA((2,)),
        pltpu.SemaphoreType.DMA((2,)),
    ],
)
def _sc_spmm(featn_hbm, srcg_hbm, dstr_hbm, zrows_hbm, part_hbm,
             sbuf, didx, rows, accum, gsem, isem):
    c = lax.axis_index("c")
    s = lax.axis_index("s")
    wid = s * NC + c
    base = s * RPT

    def start_idx(g, gb):
        pltpu.async_copy(srcg_hbm.at[wid, g], sbuf.at[gb], isem.at[gb])

    def wait_idx(g, gb):
        pltpu.make_async_copy(srcg_hbm.at[wid, g], sbuf.at[gb],
                              isem.at[gb]).wait()

    def start_gather(gb, k, rb):
        pltpu.async_copy(featn_hbm.at[sbuf.at[gb, k]], rows.at[rb],
                         gsem.at[rb])

    def wait_gather(gb, k, rb):
        pltpu.make_async_copy(featn_hbm.at[sbuf.at[gb, k]], rows.at[rb],
                              gsem.at[rb]).wait()

    pltpu.sync_copy(zrows_hbm, accum.at[pl.ds(base, RPT)])
    pltpu.sync_copy(dstr_hbm.at[wid], didx)
    start_idx(0, 0)
    plsc.subcore_barrier()

    wait_idx(0, 0)
    start_idx(1, 1)
    start_gather(0, 0, 0)

    @pl.loop(0, NCHB)
    def _chunk(t):
        rb = lax.rem(t, 2)
        g = lax.div(t, G)
        k = lax.rem(t, G)
        gb = lax.rem(g, 2)
        tn = t + 1
        gn = lax.div(tn, G)
        kn = lax.rem(tn, G)
        rbn = lax.rem(tn, 2)
        gbn = lax.rem(gn, 2)

        @pl.when(jnp.logical_and(tn < NCHB, kn == 0))
        def _():
            wait_idx(gn, gbn)

            @pl.when(gn + 1 < NG)
            def _():
                start_idx(gn + 1, 1 - gbn)

        @pl.when(tn < NCHB)
        def _():
            start_gather(gbn, kn, rbn)

        wait_gather(gb, k, rb)
        pltpu.sync_copy(rows.at[rb], accum.at[didx.at[t]], add=True)

    plsc.subcore_barrier()
    pltpu.sync_copy(accum.at[pl.ds(base, RPT)], part_hbm.at[c, pl.ds(base, RPT)])


def _tc_scale(feat, deg):
    def body(f_ref, d_ref, o_ref):
        o_ref[...] = f_ref[...] * lax.rsqrt(jnp.maximum(d_ref[...], 1.0))

    blk = 1000
    return pl.pallas_call(
        body,
        grid=(N // blk,),
        in_specs=[
            pl.BlockSpec((blk, D), lambda i: (i, 0)),
            pl.BlockSpec((blk, 1), lambda i: (i, 0)),
        ],
        out_specs=pl.BlockSpec((blk, D), lambda i: (i, 0)),
        out_shape=jax.ShapeDtypeStruct((N, D), jnp.float32),
    )(feat, deg)


def _tc_matmul_scale(parts, weight, deg):
    def body(p_ref, w_ref, d_ref, o_ref):
        ssum = p_ref[0] + p_ref[1]
        r = jnp.dot(ssum, w_ref[...], preferred_element_type=jnp.float32)
        o_ref[...] = r * lax.rsqrt(jnp.maximum(d_ref[...], 1.0))

    blk = 1000
    return pl.pallas_call(
        body,
        grid=(N // blk,),
        in_specs=[
            pl.BlockSpec((NC, blk, D), lambda i: (0, i, 0)),
            pl.BlockSpec((D, D), lambda i: (0, 0)),
            pl.BlockSpec((blk, 1), lambda i: (i, 0)),
        ],
        out_specs=pl.BlockSpec((blk, D), lambda i: (i, 0)),
        out_shape=jax.ShapeDtypeStruct((N, D), jnp.float32),
    )(parts, weight, deg)


def kernel(feat, edge_index, weight):
    src = edge_index[0].astype(jnp.int32)
    dst = edge_index[1].astype(jnp.int32)
    pad = EPWP - EPW
    srcp = jnp.pad(src.reshape(NW, EPW), ((0, 0), (0, pad)),
                   constant_values=0)
    dstp = jnp.pad(dst.reshape(NW, EPW), ((0, 0), (0, pad)),
                   constant_values=NP - 1).reshape(NW, NCHB, CHB)

    ones1 = jnp.ones((CHB,), jnp.float32)
    zeros1 = jnp.zeros((RPD,), jnp.float32)
    degp = _sc_degrees(srcp.reshape(NW, NCHB, CHB), dstp, ones1, zeros1)
    deg_src = (degp[0, 0, :N] + degp[1, 0, :N]).at[0].add(-NW * pad)
    deg_src = deg_src.reshape(N, 1)
    deg_dst = (degp[0, 1, :N] + degp[1, 1, :N]).reshape(N, 1)

    featn = _tc_scale(feat, deg_src)

    srcg = srcp.reshape(NW, NG, G, CHB)
    zrows = jnp.zeros((RPT, D), jnp.float32)
    parts = _sc_spmm(featn, srcg, dstp, zrows)

    return _tc_matmul_scale(parts, weight, deg_dst)
